# Initial kernel scaffold; baseline (speedup 1.0000x reference)
import jax
import jax.numpy as jnp
from jax.experimental import pallas as pl


def kernel(x, conv_w, conv_b, gamma, beta, cond, cond_w, cond_bias):
    raise NotImplementedError("write your pallas kernel here")


# speedup vs baseline: 2.6924x; 2.6924x over previous
import functools

import jax
import jax.numpy as jnp
from jax import lax
from jax.experimental import pallas as pl
from jax.experimental.pallas import tpu as pltpu


def _film_conv_row(x_ref, ca_ref, cb_ref, w_ref, *, K, d, pad):
    x = x_ref[0]
    xmod = ca_ref[0] * x + cb_ref[0]
    xb = xmod.astype(jnp.bfloat16)
    C, L = xb.shape
    if pad > 0:
        z = jnp.zeros((C, pad), jnp.bfloat16)
        xp = jnp.concatenate([z, xb, z], axis=1)
    else:
        xp = xb
    y = jnp.dot(w_ref[0], xp[:, 0:L], preferred_element_type=jnp.float32)
    for k in range(1, K):
        y = y + jnp.dot(w_ref[k], xp[:, k * d:k * d + L],
                        preferred_element_type=jnp.float32)
    return xmod, y


def _stats_kernel(x_ref, ca_ref, cb_ref, w_ref, sum_ref, sq_ref, *, K, d, pad):
    i = pl.program_id(1)

    @pl.when(i == 0)
    def _init():
        sum_ref[...] = jnp.zeros_like(sum_ref)
        sq_ref[...] = jnp.zeros_like(sq_ref)

    _, y = _film_conv_row(x_ref, ca_ref, cb_ref, w_ref, K=K, d=d, pad=pad)
    sum_ref[0] += jnp.sum(y, axis=1, keepdims=True)
    sq_ref[0] += jnp.sum(y * y, axis=1, keepdims=True)


def _apply_kernel(x_ref, ca_ref, cb_ref, w_ref, scale_ref, shift_ref, out_ref,
                  *, K, d, pad):
    xmod, y = _film_conv_row(x_ref, ca_ref, cb_ref, w_ref, K=K, d=d, pad=pad)
    y = y * scale_ref[...] + shift_ref[...]
    y = jnp.maximum(y, 0.0)
    out_ref[0] = (xmod + y).astype(out_ref.dtype)


def kernel(x, conv_w, conv_b, gamma, beta, cond, cond_w, cond_bias):
    del conv_b
    dilation, eps = 2, 1e-5
    N, C, L = x.shape
    K = conv_w.shape[-1]
    d = int(dilation)
    pad = (K - 1) // 2 * d
    dt = x.dtype

    z = jax.nn.relu(cond @ cond_w[:, :, 0].T + cond_bias)
    cond_b_term = z[:, :C].reshape(N, C, 1).astype(dt)
    cond_a_term = z[:, C:].reshape(N, C, 1).astype(dt)

    w_taps = jnp.transpose(conv_w, (2, 0, 1)).astype(jnp.bfloat16)

    kcommon = dict(K=K, d=d, pad=pad)
    cparams = dict(vmem_limit_bytes=64 * 1024 * 1024)

    G = 2 if (N % 2 == 0 and N > 1) else 1
    npg = N // G

    row_spec1 = pl.BlockSpec((1, C, L), lambda g, i: (g * npg + i, 0, 0))
    cvec_spec1 = pl.BlockSpec((1, C, 1), lambda g, i: (g * npg + i, 0, 0))
    w_spec1 = pl.BlockSpec((K, C, C), lambda g, i: (0, 0, 0))
    stat_spec1 = pl.BlockSpec((1, C, 1), lambda g, i: (g, 0, 0))

    psum, psq = pl.pallas_call(
        functools.partial(_stats_kernel, **kcommon),
        out_shape=(jax.ShapeDtypeStruct((G, C, 1), jnp.float32),
                   jax.ShapeDtypeStruct((G, C, 1), jnp.float32)),
        grid=(G, npg),
        in_specs=[row_spec1, cvec_spec1, cvec_spec1, w_spec1],
        out_specs=(stat_spec1, stat_spec1),
        compiler_params=pltpu.CompilerParams(
            dimension_semantics=("parallel", "arbitrary"), **cparams),
    )(x, cond_a_term, cond_b_term, w_taps)

    cnt = jnp.float32(N * L)
    mean = jnp.sum(psum, axis=0)[:, 0] / cnt
    ex2 = jnp.sum(psq, axis=0)[:, 0] / cnt
    var = jnp.maximum(ex2 - mean * mean, 0.0)
    rstd = lax.rsqrt(var + eps)
    g32 = gamma.astype(jnp.float32)
    bn_scale = (g32 * rstd).reshape(C, 1)
    bn_shift = (beta.astype(jnp.float32) - g32 * rstd * mean).reshape(C, 1)

    row_spec2 = pl.BlockSpec((1, C, L), lambda n: (n, 0, 0))
    cvec_spec2 = pl.BlockSpec((1, C, 1), lambda n: (n, 0, 0))
    w_spec2 = pl.BlockSpec((K, C, C), lambda n: (0, 0, 0))
    col_spec2 = pl.BlockSpec((C, 1), lambda n: (0, 0))

    out = pl.pallas_call(
        functools.partial(_apply_kernel, **kcommon),
        out_shape=jax.ShapeDtypeStruct((N, C, L), dt),
        grid=(N,),
        in_specs=[row_spec2, cvec_spec2, cvec_spec2, w_spec2,
                  col_spec2, col_spec2],
        out_specs=row_spec2,
        compiler_params=pltpu.CompilerParams(
            dimension_semantics=("parallel",), **cparams),
    )(x, cond_a_term, cond_b_term, w_taps, bn_scale, bn_shift)

    return out


# speedup vs baseline: 2.8215x; 1.0479x over previous
import functools

import jax
import jax.numpy as jnp
from jax import lax
from jax.experimental import pallas as pl
from jax.experimental.pallas import tpu as pltpu


def _conv_row(xb, w_ref, *, K, d, pad):
    C, L = xb.shape
    if pad > 0:
        z = jnp.zeros((C, pad), jnp.bfloat16)
        xp = jnp.concatenate([z, xb, z], axis=1)
    else:
        xp = xb
    y = jnp.dot(w_ref[0], xp[:, 0:L], preferred_element_type=jnp.float32)
    for k in range(1, K):
        y = y + jnp.dot(w_ref[k], xp[:, k * d:k * d + L],
                        preferred_element_type=jnp.float32)
    return y


def _stats_kernel(x_ref, ca_ref, cb_ref, w_ref, xmod_ref, sum_ref, sq_ref,
                  *, K, d, pad):
    i = pl.program_id(1)

    @pl.when(i == 0)
    def _init():
        sum_ref[...] = jnp.zeros_like(sum_ref)
        sq_ref[...] = jnp.zeros_like(sq_ref)

    xmod = ca_ref[0] * x_ref[0] + cb_ref[0]
    xb = xmod.astype(jnp.bfloat16)
    xmod_ref[0] = xb
    y = _conv_row(xb, w_ref, K=K, d=d, pad=pad)
    sum_ref[0] += jnp.sum(y, axis=1, keepdims=True)
    sq_ref[0] += jnp.sum(y * y, axis=1, keepdims=True)


def _apply_kernel(xmod_ref, w_ref, scale_ref, shift_ref, out_ref,
                  *, K, d, pad):
    xb = xmod_ref[0]
    y = _conv_row(xb, w_ref, K=K, d=d, pad=pad)
    y = y * scale_ref[...] + shift_ref[...]
    y = jnp.maximum(y, 0.0)
    out_ref[0] = (xb.astype(jnp.float32) + y).astype(out_ref.dtype)


def kernel(x, conv_w, conv_b, gamma, beta, cond, cond_w, cond_bias):
    del conv_b
    dilation, eps = 2, 1e-5
    N, C, L = x.shape
    K = conv_w.shape[-1]
    d = int(dilation)
    pad = (K - 1) // 2 * d
    dt = x.dtype

    z = jax.nn.relu(cond @ cond_w[:, :, 0].T + cond_bias)
    cond_b_term = z[:, :C].reshape(N, C, 1).astype(dt)
    cond_a_term = z[:, C:].reshape(N, C, 1).astype(dt)

    w_taps = jnp.transpose(conv_w, (2, 0, 1)).astype(jnp.bfloat16)

    kcommon = dict(K=K, d=d, pad=pad)
    cparams = dict(vmem_limit_bytes=64 * 1024 * 1024)

    G = 2 if (N % 2 == 0 and N > 1) else 1
    npg = N // G

    row_spec1 = pl.BlockSpec((1, C, L), lambda g, i: (g * npg + i, 0, 0))
    cvec_spec1 = pl.BlockSpec((1, C, 1), lambda g, i: (g * npg + i, 0, 0))
    w_spec1 = pl.BlockSpec((K, C, C), lambda g, i: (0, 0, 0))
    stat_spec1 = pl.BlockSpec((1, C, 1), lambda g, i: (g, 0, 0))

    xmod_bf, psum, psq = pl.pallas_call(
        functools.partial(_stats_kernel, **kcommon),
        out_shape=(jax.ShapeDtypeStruct((N, C, L), jnp.bfloat16),
                   jax.ShapeDtypeStruct((G, C, 1), jnp.float32),
                   jax.ShapeDtypeStruct((G, C, 1), jnp.float32)),
        grid=(G, npg),
        in_specs=[row_spec1, cvec_spec1, cvec_spec1, w_spec1],
        out_specs=(row_spec1, stat_spec1, stat_spec1),
        compiler_params=pltpu.CompilerParams(
            dimension_semantics=("parallel", "arbitrary"), **cparams),
    )(x, cond_a_term, cond_b_term, w_taps)

    cnt = jnp.float32(N * L)
    mean = jnp.sum(psum, axis=0)[:, 0] / cnt
    ex2 = jnp.sum(psq, axis=0)[:, 0] / cnt
    var = jnp.maximum(ex2 - mean * mean, 0.0)
    rstd = lax.rsqrt(var + eps)
    g32 = gamma.astype(jnp.float32)
    bn_scale = (g32 * rstd).reshape(C, 1)
    bn_shift = (beta.astype(jnp.float32) - g32 * rstd * mean).reshape(C, 1)

    row_spec2 = pl.BlockSpec((1, C, L), lambda n: (n, 0, 0))
    w_spec2 = pl.BlockSpec((K, C, C), lambda n: (0, 0, 0))
    col_spec2 = pl.BlockSpec((C, 1), lambda n: (0, 0))

    out = pl.pallas_call(
        functools.partial(_apply_kernel, **kcommon),
        out_shape=jax.ShapeDtypeStruct((N, C, L), dt),
        grid=(N,),
        in_specs=[row_spec2, w_spec2, col_spec2, col_spec2],
        out_specs=row_spec2,
        compiler_params=pltpu.CompilerParams(
            dimension_semantics=("parallel",), **cparams),
    )(xmod_bf, w_taps, bn_scale, bn_shift)

    return out
